# baseline (speedup 1.0000x reference)
import functools

import jax
import jax.numpy as jnp
from jax.experimental import pallas as pl

_EPS = 1e-6


def _vq_block_kernel(x_ref, cb_ref, x2_ref, c2_ref, out_ref, idx_ref, loss_ref,
                     *, scale):
    x = x_ref[...]
    cb = cb_ref[...]
    x2 = x2_ref[...]
    c2 = c2_ref[...]

    xc = jax.lax.dot_general(
        x, cb, dimension_numbers=(((1,), (1,)), ((), ())),
        preferred_element_type=jnp.float32)
    dist = jnp.sqrt(jnp.maximum(x2 + c2 - 2.0 * xc, 0.0))

    mn = jnp.min(dist, axis=1, keepdims=True)
    iota = jax.lax.broadcasted_iota(jnp.int32, dist.shape, 1)
    idx = jnp.min(jnp.where(dist == mn, iota, jnp.int32(2**30)), axis=1)

    onehot = (iota == idx[:, None]).astype(jnp.float32)
    q = jax.lax.dot_general(
        onehot, cb, dimension_numbers=(((1,), (0,)), ((), ())),
        preferred_element_type=jnp.float32)

    ns = jnp.sqrt(x2)
    nt = jnp.sqrt(jnp.sum(q * q, axis=1, keepdims=True))
    u = x / jnp.maximum(ns, _EPS)
    qn = q / jnp.maximum(nt, _EPS)
    wv = u + qn
    w = wv / jnp.maximum(jnp.sqrt(jnp.sum(wv * wv, axis=1, keepdims=True)), _EPS)
    dew = jnp.sum(x * w, axis=1, keepdims=True)
    deu = jnp.sum(x * u, axis=1, keepdims=True)
    rot = x - 2.0 * dew * w + 2.0 * deu * qn
    out_ref[...] = rot * (nt / jnp.maximum(ns, _EPS))

    idx_ref[0, 0, :] = idx

    diff = x - q
    part = (jnp.sum(diff * diff) * scale).reshape(1, 1)

    @pl.when(pl.program_id(0) == 0)
    def _init():
        loss_ref[...] = part

    @pl.when(pl.program_id(0) != 0)
    def _acc():
        loss_ref[...] += part


def kernel(x, codebook):
    n, d = x.shape
    k = codebook.shape[0]
    block = 2048
    nb = n // block

    x2 = jnp.sum(x * x, axis=-1, keepdims=True)
    c2 = jnp.sum(codebook * codebook, axis=-1)[None, :]

    out, idx3, loss = pl.pallas_call(
        functools.partial(_vq_block_kernel, scale=1.25 / (n * d)),
        grid=(nb,),
        in_specs=[
            pl.BlockSpec((block, d), lambda i: (i, 0)),
            pl.BlockSpec((k, d), lambda i: (0, 0)),
            pl.BlockSpec((block, 1), lambda i: (i, 0)),
            pl.BlockSpec((1, k), lambda i: (0, 0)),
        ],
        out_specs=[
            pl.BlockSpec((block, d), lambda i: (i, 0)),
            pl.BlockSpec((1, 1, block), lambda i: (i, 0, 0)),
            pl.BlockSpec((1, 1), lambda i: (0, 0)),
        ],
        out_shape=[
            jax.ShapeDtypeStruct((n, d), jnp.float32),
            jax.ShapeDtypeStruct((nb, 1, block), jnp.int32),
            jax.ShapeDtypeStruct((1, 1), jnp.float32),
        ],
    )(x, codebook, x2, c2)

    return out, idx3.reshape(n), loss[0, 0]
